# baseline (speedup 1.0000x reference)
import functools

import jax
import jax.numpy as jnp
from jax import lax
from jax.experimental import pallas as pl
from jax.experimental.pallas import tpu as pltpu
from jax.experimental.pallas import tpu_sc as plsc

DIM = 1024
SEQ = 8192
BATCH = 2

NUM_WORKERS = 32
ROWS_PER_W = SEQ // NUM_WORKERS
CHUNK = 32
NCHUNK = ROWS_PER_W // CHUNK

ACHUNK = 16
NACHUNK = ROWS_PER_W // ACHUNK
GROUPS = DIM // 16

BS = 512
NBLK = SEQ // BS


def _sc_gather_body(pe_hbm, pos_hbm, g_hbm, idx_v, buf0, buf1, sem0, sem1):
    wid = lax.axis_index("s") * 2 + lax.axis_index("c")
    base = wid * ROWS_PER_W
    pltpu.sync_copy(pos_hbm.at[pl.ds(base, ROWS_PER_W)], idx_v)
    bufs = (buf0, buf1)
    sems = (sem0, sem1)
    copies = []
    for c in range(NCHUNK):
        copies.append(
            pltpu.async_copy(
                pe_hbm.at[idx_v.at[pl.ds(c * CHUNK, CHUNK)]],
                bufs[c % 2],
                sems[c % 2],
            )
        )
        if c > 0:
            copies[c - 1].wait()
            pltpu.sync_copy(
                bufs[(c - 1) % 2],
                g_hbm.at[pl.ds(base + (c - 1) * CHUNK, CHUNK)],
            )
    copies[NCHUNK - 1].wait()
    pltpu.sync_copy(
        bufs[(NCHUNK - 1) % 2],
        g_hbm.at[pl.ds(base + (NCHUNK - 1) * CHUNK, CHUNK)],
    )


_gather = functools.partial(
    pl.kernel,
    out_type=jax.ShapeDtypeStruct((SEQ, DIM), jnp.float32),
    mesh=plsc.VectorSubcoreMesh(core_axis_name="c", subcore_axis_name="s"),
    scratch_types=[
        pltpu.VMEM((ROWS_PER_W,), jnp.int32),
        pltpu.VMEM((CHUNK, DIM), jnp.float32),
        pltpu.VMEM((CHUNK, DIM), jnp.float32),
        pltpu.SemaphoreType.DMA,
        pltpu.SemaphoreType.DMA,
    ],
)(_sc_gather_body)


def _sc_add_body(k_hbm, g_hbm, ok_hbm, gb0, gb1, db0, db1, db2, db3,
                 gsem0, gsem1, dsem0, dsem1, dsem2, dsem3, osem):
    wid = lax.axis_index("s") * 2 + lax.axis_index("c")
    base = wid * ROWS_PER_W
    gbufs = (gb0, gb1)
    gsems = (gsem0, gsem1)
    dbufs = (db0, db1, db2, db3)
    dsems = (dsem0, dsem1, dsem2, dsem3)

    def g_load(c):
        return pltpu.async_copy(
            g_hbm.at[pl.ds(base + c * ACHUNK, ACHUNK)], gbufs[c % 2], gsems[c % 2]
        )

    def k_load(c, b):
        j = (2 * c + b) % 4
        return pltpu.async_copy(
            k_hbm.at[pl.ds(b * SEQ + base + c * ACHUNK, ACHUNK)], dbufs[j], dsems[j]
        )

    g_copies = [g_load(0), g_load(1)]
    d_copies = [k_load(0, 0), k_load(0, 1), k_load(1, 0), k_load(1, 1)]

    for c in range(NACHUNK):
        gbuf = gbufs[c % 2]
        g_copies[c].wait()
        for b in range(2):
            j = (2 * c + b) % 4
            dbuf = dbufs[j]
            d_copies[2 * c + b].wait()

            def row_add(r, carry, dbuf=dbuf, gbuf=gbuf):
                for col in range(GROUPS):
                    s = pl.ds(col * 16, 16)
                    dbuf[r, s] = dbuf[r, s] + gbuf[r, s]
                return carry

            lax.fori_loop(0, ACHUNK, row_add, 0)
            pltpu.async_copy(
                dbuf, ok_hbm.at[pl.ds(b * SEQ + base + c * ACHUNK, ACHUNK)], osem
            ).wait()
            if c + 2 < NACHUNK:
                d_copies.append(k_load(c + 2, b))
        if c + 2 < NACHUNK:
            g_copies.append(g_load(c + 2))


_sc_add = functools.partial(
    pl.kernel,
    out_type=jax.ShapeDtypeStruct((BATCH * SEQ, DIM), jnp.float32),
    mesh=plsc.VectorSubcoreMesh(core_axis_name="c", subcore_axis_name="s"),
    scratch_types=(
        [pltpu.VMEM((ACHUNK, DIM), jnp.float32) for _ in range(6)]
        + [pltpu.SemaphoreType.DMA for _ in range(7)]
    ),
)(_sc_add_body)


def _tc_add(q_ref, g_ref, oq_ref):
    oq_ref[...] = q_ref[...] + g_ref[...][None, :, :]


_q_add = pl.pallas_call(
    _tc_add,
    grid=(NBLK,),
    in_specs=[
        pl.BlockSpec((BATCH, BS, DIM), lambda j: (0, j, 0)),
        pl.BlockSpec((BS, DIM), lambda j: (j, 0)),
    ],
    out_specs=pl.BlockSpec((BATCH, BS, DIM), lambda j: (0, j, 0)),
    out_shape=jax.ShapeDtypeStruct((BATCH, SEQ, DIM), jnp.float32),
)


@jax.jit
def kernel(q, k, pos, pe):
    g = _gather(pe, pos)
    ok = _sc_add(k.reshape(BATCH * SEQ, DIM), g)
    oq = _q_add(q, g)
    return oq, ok.reshape(BATCH, SEQ, DIM)
